# baseline (speedup 1.0000x reference)
import functools

import jax
import jax.numpy as jnp
from jax import lax
from jax.experimental import pallas as pl
from jax.experimental.pallas import tpu as pltpu
from jax.experimental.pallas import tpu_sc as plsc

N_TOKENS = 16384
D_MODEL = 2048
N_EXPERTS = 64
T_TILE = 1024

NUM_CORES = 2
NUM_SUBCORES = 16
NUM_WORKERS = NUM_CORES * NUM_SUBCORES
CHUNK = N_TOKENS // NUM_WORKERS
LANES = 16
GROUPS = CHUNK // LANES


def _logits_body(x_ref, w_ref, out_ref):
    logits = lax.dot_general(
        x_ref[...], w_ref[...],
        (((1,), (1,)), ((), ())),
        preferred_element_type=jnp.float32,
    )
    out_ref[...] = logits.T


def _compute_logits_t(x, W):
    return pl.pallas_call(
        _logits_body,
        grid=(N_TOKENS // T_TILE,),
        in_specs=[
            pl.BlockSpec((T_TILE, D_MODEL), lambda i: (i, 0)),
            pl.BlockSpec((N_EXPERTS, D_MODEL), lambda i: (0, 0)),
        ],
        out_specs=pl.BlockSpec((N_EXPERTS, T_TILE), lambda i: (0, i)),
        out_shape=jax.ShapeDtypeStruct((N_EXPERTS, N_TOKENS), jnp.float32),
        compiler_params=pltpu.CompilerParams(
            dimension_semantics=("arbitrary",),
        ),
    )(x, W)


_SC_MESH = plsc.VectorSubcoreMesh(core_axis_name="c", subcore_axis_name="s")


@functools.partial(
    pl.kernel,
    out_type=[
        jax.ShapeDtypeStruct((N_TOKENS,), jnp.int32),
        jax.ShapeDtypeStruct((N_TOKENS,), jnp.int32),
        jax.ShapeDtypeStruct((N_TOKENS,), jnp.float32),
        jax.ShapeDtypeStruct((N_TOKENS,), jnp.float32),
    ],
    mesh=_SC_MESH,
    scratch_types=[
        pltpu.VMEM((N_EXPERTS, CHUNK), jnp.float32),
        pltpu.VMEM((CHUNK,), jnp.int32),
        pltpu.VMEM((CHUNK,), jnp.int32),
        pltpu.VMEM((CHUNK,), jnp.float32),
        pltpu.VMEM((CHUNK,), jnp.float32),
    ],
)
def _sc_top2(lg_hbm, i1_hbm, i2_hbm, g1_hbm, g2_hbm,
             lg_v, i1_v, i2_v, g1_v, g2_v):
    wid = lax.axis_index("s") * NUM_CORES + lax.axis_index("c")
    base = wid * CHUNK
    pltpu.sync_copy(lg_hbm.at[:, pl.ds(base, CHUNK)], lg_v)

    def group(j, carry):
        sl = pl.ds(j * LANES, LANES)
        m1 = lg_v[0, sl]
        i1 = jnp.zeros((LANES,), jnp.int32)
        m2 = jnp.full((LANES,), -jnp.inf, jnp.float32)
        i2 = jnp.zeros((LANES,), jnp.int32)
        for e in range(1, N_EXPERTS):
            e_vec = jnp.full((LANES,), e, jnp.int32)
            v = lg_v[e, sl]
            gt1 = v > m1
            gt2 = v > m2
            i2 = jnp.where(gt1, i1, jnp.where(gt2, e_vec, i2))
            m2 = jnp.where(gt1, m1, jnp.where(gt2, v, m2))
            i1 = jnp.where(gt1, e_vec, i1)
            m1 = jnp.where(gt1, v, m1)
        t = jnp.exp(m2 - m1)
        den = 1.0 + t
        i1_v[sl] = i1
        i2_v[sl] = i2
        g1_v[sl] = 1.0 / den
        g2_v[sl] = t / den
        return carry

    lax.fori_loop(0, GROUPS, group, 0)
    pltpu.sync_copy(i1_v, i1_hbm.at[pl.ds(base, CHUNK)])
    pltpu.sync_copy(i2_v, i2_hbm.at[pl.ds(base, CHUNK)])
    pltpu.sync_copy(g1_v, g1_hbm.at[pl.ds(base, CHUNK)])
    pltpu.sync_copy(g2_v, g2_hbm.at[pl.ds(base, CHUNK)])


@jax.jit
def kernel(x, W):
    lg_t = _compute_logits_t(x, W)
    i1, i2, g1, g2 = _sc_top2(lg_t)
    idx = jnp.stack([i1, i2], axis=-1)
    gates = jnp.stack([g1, g2], axis=-1)
    return (idx, gates)


# speedup vs baseline: 1.0110x; 1.0110x over previous
import functools

import jax
import jax.numpy as jnp
from jax import lax
from jax.experimental import pallas as pl
from jax.experimental.pallas import tpu as pltpu
from jax.experimental.pallas import tpu_sc as plsc

N_TOKENS = 16384
D_MODEL = 2048
N_EXPERTS = 64
T_TILE = 1024

NUM_CORES = 2
NUM_SUBCORES = 16
NUM_WORKERS = NUM_CORES * NUM_SUBCORES
CHUNK = N_TOKENS // NUM_WORKERS
LANES = 16
GROUPS = CHUNK // LANES


def _logits_body(x_ref, w_ref, out_ref):
    logits = lax.dot_general(
        x_ref[...], w_ref[...],
        (((1,), (1,)), ((), ())),
        preferred_element_type=jnp.float32,
    )
    out_ref[...] = logits.T


def _compute_logits_t(x, W):
    return pl.pallas_call(
        _logits_body,
        grid=(N_TOKENS // T_TILE,),
        in_specs=[
            pl.BlockSpec((T_TILE, D_MODEL), lambda i: (i, 0)),
            pl.BlockSpec((N_EXPERTS, D_MODEL), lambda i: (0, 0)),
        ],
        out_specs=pl.BlockSpec((N_EXPERTS, T_TILE), lambda i: (0, i)),
        out_shape=jax.ShapeDtypeStruct((N_EXPERTS, N_TOKENS), jnp.float32),
        compiler_params=pltpu.CompilerParams(
            dimension_semantics=("arbitrary",),
        ),
    )(x, W)


_SC_MESH = plsc.VectorSubcoreMesh(core_axis_name="c", subcore_axis_name="s")


@functools.partial(
    pl.kernel,
    out_type=[
        jax.ShapeDtypeStruct((N_TOKENS,), jnp.int32),
        jax.ShapeDtypeStruct((N_TOKENS,), jnp.int32),
        jax.ShapeDtypeStruct((N_TOKENS,), jnp.float32),
        jax.ShapeDtypeStruct((N_TOKENS,), jnp.float32),
    ],
    mesh=_SC_MESH,
    scratch_types=[
        pltpu.VMEM((N_EXPERTS, CHUNK), jnp.float32),
        pltpu.VMEM((CHUNK,), jnp.int32),
        pltpu.VMEM((CHUNK,), jnp.int32),
        pltpu.VMEM((CHUNK,), jnp.float32),
        pltpu.VMEM((CHUNK,), jnp.float32),
    ],
)
def _sc_top2(lg_hbm, i1_hbm, i2_hbm, g1_hbm, g2_hbm,
             lg_v, i1_v, i2_v, g1_v, g2_v):
    wid = lax.axis_index("s") * NUM_CORES + lax.axis_index("c")
    base = wid * CHUNK
    pltpu.sync_copy(lg_hbm.at[:, pl.ds(base, CHUNK)], lg_v)

    UNROLL = 4

    def top2_one_group(sl):
        m1 = lg_v[0, sl]
        i1 = jnp.zeros((LANES,), jnp.int32)
        m2 = jnp.full((LANES,), -jnp.inf, jnp.float32)
        i2 = jnp.zeros((LANES,), jnp.int32)
        for e in range(1, N_EXPERTS):
            e_vec = jnp.full((LANES,), e, jnp.int32)
            v = lg_v[e, sl]
            gt1 = v > m1
            gt2 = v > m2
            i2 = jnp.where(gt1, i1, jnp.where(gt2, e_vec, i2))
            m2 = jnp.maximum(m2, jnp.minimum(v, m1))
            i1 = jnp.where(gt1, e_vec, i1)
            m1 = jnp.maximum(m1, v)
        return m1, i1, m2, i2

    def group(j, carry):
        for u in range(UNROLL):
            sl = pl.ds((j * UNROLL + u) * LANES, LANES)
            m1, i1, m2, i2 = top2_one_group(sl)
            t = jnp.exp(m2 - m1)
            den = 1.0 + t
            i1_v[sl] = i1
            i2_v[sl] = i2
            g1_v[sl] = 1.0 / den
            g2_v[sl] = t / den
        return carry

    lax.fori_loop(0, GROUPS // UNROLL, group, 0)
    pltpu.sync_copy(i1_v, i1_hbm.at[pl.ds(base, CHUNK)])
    pltpu.sync_copy(i2_v, i2_hbm.at[pl.ds(base, CHUNK)])
    pltpu.sync_copy(g1_v, g1_hbm.at[pl.ds(base, CHUNK)])
    pltpu.sync_copy(g2_v, g2_hbm.at[pl.ds(base, CHUNK)])


@jax.jit
def kernel(x, W):
    lg_t = _compute_logits_t(x, W)
    i1, i2, g1, g2 = _sc_top2(lg_t)
    idx = jnp.stack([i1, i2], axis=-1)
    gates = jnp.stack([g1, g2], axis=-1)
    return (idx, gates)
